# initial kernel scaffold (unmeasured)
import jax
import jax.numpy as jnp
from jax import lax
from jax.experimental import pallas as pl
from jax.experimental.pallas import tpu as pltpu


def kernel(
    x,
):
    def body(*refs):
        pass

    out_shape = jax.ShapeDtypeStruct(..., jnp.float32)
    return pl.pallas_call(body, out_shape=out_shape)(...)


# baseline (device time: 779585 ns/iter reference)
import jax
import jax.numpy as jnp
from jax import lax
from jax.experimental import pallas as pl
from jax.experimental.pallas import tpu as pltpu

NC = 8


def kernel(x):
    m, n = x.shape
    ch = m // NC

    def body(x_ref, out_ref, recv_ref, a_ref, b_ref, o_ref,
             send_sems, recv_sems, cp_sem):
        my_x = lax.axis_index("x")
        my_y = lax.axis_index("y")
        my_z = lax.axis_index("z")
        partner = (my_x, my_y, 1 - my_z)

        barrier = pltpu.get_barrier_semaphore()
        pl.semaphore_signal(barrier, inc=1, device_id=partner,
                            device_id_type=pl.DeviceIdType.MESH)
        pl.semaphore_wait(barrier, 1)

        rdmas = []
        for c in range(NC):
            rdma = pltpu.make_async_remote_copy(
                src_ref=x_ref.at[pl.ds(c * ch, ch), :],
                dst_ref=recv_ref.at[pl.ds(c * ch, ch), :],
                send_sem=send_sems.at[c],
                recv_sem=recv_sems.at[c],
                device_id=partner,
                device_id_type=pl.DeviceIdType.MESH,
            )
            rdma.start()
            rdmas.append(rdma)

        for c in range(NC):
            rdmas[c].wait_recv()
            cp_a = pltpu.make_async_copy(
                x_ref.at[pl.ds(c * ch, ch), :], a_ref, cp_sem)
            cp_a.start()
            cp_a.wait()
            cp_b = pltpu.make_async_copy(
                recv_ref.at[pl.ds(c * ch, ch), :], b_ref, cp_sem)
            cp_b.start()
            cp_b.wait()
            o_ref[...] = a_ref[...] + b_ref[...]
            cp_o = pltpu.make_async_copy(
                o_ref, out_ref.at[pl.ds(c * ch, ch), :], cp_sem)
            cp_o.start()
            cp_o.wait()

        for c in range(NC):
            rdmas[c].wait_send()

    out, _ = pl.pallas_call(
        body,
        out_shape=(
            jax.ShapeDtypeStruct((m, n), x.dtype),
            jax.ShapeDtypeStruct((m, n), x.dtype),
        ),
        in_specs=[pl.BlockSpec(memory_space=pl.ANY)],
        out_specs=(
            pl.BlockSpec(memory_space=pl.ANY),
            pl.BlockSpec(memory_space=pl.ANY),
        ),
        scratch_shapes=[
            pltpu.VMEM((ch, n), x.dtype),
            pltpu.VMEM((ch, n), x.dtype),
            pltpu.VMEM((ch, n), x.dtype),
            pltpu.SemaphoreType.DMA((NC,)),
            pltpu.SemaphoreType.DMA((NC,)),
            pltpu.SemaphoreType.DMA,
        ],
        compiler_params=pltpu.CompilerParams(collective_id=0),
    )(x)
    return out


# device time: 373372 ns/iter; 2.0880x vs baseline; 2.0880x over previous
import jax
import jax.numpy as jnp
from jax import lax
from jax.experimental import pallas as pl
from jax.experimental.pallas import tpu as pltpu

KC = 4


def kernel(x):
    m, n = x.shape
    blk = m // 4
    cr = blk // KC

    def body(x_ref, out_ref, zrecv_ref, a_ref, o_ref,
             sz_send, sz_recv, sx_send, sx_recv, sy_send, sy_recv,
             srx_send, srx_recv, sry_send, sry_recv, cp_sem):
        my_x = lax.axis_index("x")
        my_y = lax.axis_index("y")
        my_z = lax.axis_index("z")
        z_nbr = (my_x, my_y, 1 - my_z)
        x_nbr = (1 - my_x, my_y, my_z)
        y_nbr = (my_x, 1 - my_y, my_z)

        r = 2 * my_x + my_y
        rx = 2 * (1 - my_x) + my_y
        ry = 2 * my_x + (1 - my_y)
        rd = 2 * (1 - my_x) + (1 - my_y)

        def rows(blk_id, k):
            return pl.ds(blk_id * blk + k * cr, cr)

        barrier = pltpu.get_barrier_semaphore()
        for nbr in (z_nbr, x_nbr, y_nbr):
            pl.semaphore_signal(barrier, inc=1, device_id=nbr,
                                device_id_type=pl.DeviceIdType.MESH)
        pl.semaphore_wait(barrier, 3)

        z_rdmas = []
        for k in range(KC):
            rdma = pltpu.make_async_remote_copy(
                src_ref=x_ref.at[rows(r, k), :],
                dst_ref=zrecv_ref.at[pl.ds(k * cr, cr), :],
                send_sem=sz_send.at[k],
                recv_sem=sz_recv.at[k],
                device_id=z_nbr,
                device_id_type=pl.DeviceIdType.MESH,
            )
            rdma.start()
            z_rdmas.append(rdma)

        def recv_desc(blk_id, k, send_sem, recv_sem, nbr):
            return pltpu.make_async_remote_copy(
                src_ref=out_ref.at[rows(blk_id, k), :],
                dst_ref=out_ref.at[rows(blk_id, k), :],
                send_sem=send_sem.at[k],
                recv_sem=recv_sem.at[k],
                device_id=nbr,
                device_id_type=pl.DeviceIdType.MESH,
            )

        xy_rdmas = []
        for k in range(KC):
            z_rdmas[k].wait_recv()
            cp_a = pltpu.make_async_copy(
                x_ref.at[rows(r, k), :], a_ref, cp_sem)
            cp_a.start()
            cp_a.wait()
            o_ref[...] = a_ref[...] + zrecv_ref[pl.ds(k * cr, cr), :]
            cp_o = pltpu.make_async_copy(
                o_ref, out_ref.at[rows(r, k), :], cp_sem)
            cp_o.start()
            cp_o.wait()
            for send_sem, recv_sem, nbr in (
                (sx_send, sx_recv, x_nbr),
                (sy_send, sy_recv, y_nbr),
            ):
                rdma = pltpu.make_async_remote_copy(
                    src_ref=out_ref.at[rows(r, k), :],
                    dst_ref=out_ref.at[rows(r, k), :],
                    send_sem=send_sem.at[k],
                    recv_sem=recv_sem.at[k],
                    device_id=nbr,
                    device_id_type=pl.DeviceIdType.MESH,
                )
                rdma.start()
                xy_rdmas.append(rdma)

        x_in = [recv_desc(rx, k, sx_send, sx_recv, x_nbr) for k in range(KC)]
        y_in = [recv_desc(ry, k, sy_send, sy_recv, y_nbr) for k in range(KC)]
        d_in = [
            recv_desc(rd, k,
                      srx_send if k % 2 == 0 else sry_send,
                      srx_recv if k % 2 == 0 else sry_recv,
                      x_nbr if k % 2 == 0 else y_nbr)
            for k in range(KC)
        ]
        relays = []
        for k in range(KC):
            if k % 2 == 0:
                y_in[k].wait_recv()
                src_blk, send_sem, recv_sem, nbr = ry, srx_send, srx_recv, x_nbr
            else:
                x_in[k].wait_recv()
                src_blk, send_sem, recv_sem, nbr = rx, sry_send, sry_recv, y_nbr
            rdma = pltpu.make_async_remote_copy(
                src_ref=out_ref.at[rows(src_blk, k), :],
                dst_ref=out_ref.at[rows(src_blk, k), :],
                send_sem=send_sem.at[k],
                recv_sem=recv_sem.at[k],
                device_id=nbr,
                device_id_type=pl.DeviceIdType.MESH,
            )
            rdma.start()
            relays.append(rdma)

        for k in range(KC):
            if k % 2 == 0:
                x_in[k].wait_recv()
            else:
                y_in[k].wait_recv()
            d_in[k].wait_recv()
        for rdma in z_rdmas + xy_rdmas + relays:
            rdma.wait_send()

    return pl.pallas_call(
        body,
        out_shape=jax.ShapeDtypeStruct((m, n), x.dtype),
        in_specs=[pl.BlockSpec(memory_space=pl.ANY)],
        out_specs=pl.BlockSpec(memory_space=pl.ANY),
        scratch_shapes=[
            pltpu.VMEM((blk, n), x.dtype),
            pltpu.VMEM((cr, n), x.dtype),
            pltpu.VMEM((cr, n), x.dtype),
            pltpu.SemaphoreType.DMA((KC,)),
            pltpu.SemaphoreType.DMA((KC,)),
            pltpu.SemaphoreType.DMA((KC,)),
            pltpu.SemaphoreType.DMA((KC,)),
            pltpu.SemaphoreType.DMA((KC,)),
            pltpu.SemaphoreType.DMA((KC,)),
            pltpu.SemaphoreType.DMA((KC,)),
            pltpu.SemaphoreType.DMA((KC,)),
            pltpu.SemaphoreType.DMA((KC,)),
            pltpu.SemaphoreType.DMA((KC,)),
            pltpu.SemaphoreType.DMA,
        ],
        compiler_params=pltpu.CompilerParams(collective_id=0),
    )(x)


# device time: 348642 ns/iter; 2.2361x vs baseline; 1.0709x over previous
import jax
import jax.numpy as jnp
from jax import lax
from jax.experimental import pallas as pl
from jax.experimental.pallas import tpu as pltpu

KC = 8


def kernel(x):
    m, n = x.shape
    blk = m // 4
    cr = blk // KC

    def body(x_ref, out_ref, zrecv_ref, a_ref, o_ref,
             sz_send, sz_recv, sx_send, sx_recv, sy_send, sy_recv,
             srx_send, srx_recv, sry_send, sry_recv, cp_sem):
        my_x = lax.axis_index("x")
        my_y = lax.axis_index("y")
        my_z = lax.axis_index("z")
        z_nbr = (my_x, my_y, 1 - my_z)
        x_nbr = (1 - my_x, my_y, my_z)
        y_nbr = (my_x, 1 - my_y, my_z)

        r = 2 * my_x + my_y
        rx = 2 * (1 - my_x) + my_y
        ry = 2 * my_x + (1 - my_y)
        rd = 2 * (1 - my_x) + (1 - my_y)

        def rows(blk_id, k):
            return pl.ds(blk_id * blk + k * cr, cr)

        barrier = pltpu.get_barrier_semaphore()
        for nbr in (z_nbr, x_nbr, y_nbr):
            pl.semaphore_signal(barrier, inc=1, device_id=nbr,
                                device_id_type=pl.DeviceIdType.MESH)
        pl.semaphore_wait(barrier, 3)

        z_rdmas = []
        for k in range(KC):
            rdma = pltpu.make_async_remote_copy(
                src_ref=x_ref.at[rows(r, k), :],
                dst_ref=zrecv_ref.at[pl.ds(k * cr, cr), :],
                send_sem=sz_send.at[k],
                recv_sem=sz_recv.at[k],
                device_id=z_nbr,
                device_id_type=pl.DeviceIdType.MESH,
            )
            rdma.start()
            z_rdmas.append(rdma)

        def recv_desc(blk_id, k, send_sem, recv_sem, nbr):
            return pltpu.make_async_remote_copy(
                src_ref=out_ref.at[rows(blk_id, k), :],
                dst_ref=out_ref.at[rows(blk_id, k), :],
                send_sem=send_sem.at[k],
                recv_sem=recv_sem.at[k],
                device_id=nbr,
                device_id_type=pl.DeviceIdType.MESH,
            )

        xy_rdmas = []
        for k in range(KC):
            z_rdmas[k].wait_recv()
            cp_a = pltpu.make_async_copy(
                x_ref.at[rows(r, k), :], a_ref, cp_sem)
            cp_a.start()
            cp_a.wait()
            o_ref[...] = a_ref[...] + zrecv_ref[pl.ds(k * cr, cr), :]
            cp_o = pltpu.make_async_copy(
                o_ref, out_ref.at[rows(r, k), :], cp_sem)
            cp_o.start()
            cp_o.wait()
            for send_sem, recv_sem, nbr in (
                (sx_send, sx_recv, x_nbr),
                (sy_send, sy_recv, y_nbr),
            ):
                rdma = pltpu.make_async_remote_copy(
                    src_ref=out_ref.at[rows(r, k), :],
                    dst_ref=out_ref.at[rows(r, k), :],
                    send_sem=send_sem.at[k],
                    recv_sem=recv_sem.at[k],
                    device_id=nbr,
                    device_id_type=pl.DeviceIdType.MESH,
                )
                rdma.start()
                xy_rdmas.append(rdma)

        x_in = [recv_desc(rx, k, sx_send, sx_recv, x_nbr) for k in range(KC)]
        y_in = [recv_desc(ry, k, sy_send, sy_recv, y_nbr) for k in range(KC)]
        d_in = [
            recv_desc(rd, k,
                      srx_send if k % 2 == 0 else sry_send,
                      srx_recv if k % 2 == 0 else sry_recv,
                      x_nbr if k % 2 == 0 else y_nbr)
            for k in range(KC)
        ]
        relays = []
        for k in range(KC):
            if k % 2 == 0:
                y_in[k].wait_recv()
                src_blk, send_sem, recv_sem, nbr = ry, srx_send, srx_recv, x_nbr
            else:
                x_in[k].wait_recv()
                src_blk, send_sem, recv_sem, nbr = rx, sry_send, sry_recv, y_nbr
            rdma = pltpu.make_async_remote_copy(
                src_ref=out_ref.at[rows(src_blk, k), :],
                dst_ref=out_ref.at[rows(src_blk, k), :],
                send_sem=send_sem.at[k],
                recv_sem=recv_sem.at[k],
                device_id=nbr,
                device_id_type=pl.DeviceIdType.MESH,
            )
            rdma.start()
            relays.append(rdma)

        for k in range(KC):
            if k % 2 == 0:
                x_in[k].wait_recv()
            else:
                y_in[k].wait_recv()
            d_in[k].wait_recv()
        for rdma in z_rdmas + xy_rdmas + relays:
            rdma.wait_send()

    return pl.pallas_call(
        body,
        out_shape=jax.ShapeDtypeStruct((m, n), x.dtype),
        in_specs=[pl.BlockSpec(memory_space=pl.ANY)],
        out_specs=pl.BlockSpec(memory_space=pl.ANY),
        scratch_shapes=[
            pltpu.VMEM((blk, n), x.dtype),
            pltpu.VMEM((cr, n), x.dtype),
            pltpu.VMEM((cr, n), x.dtype),
            pltpu.SemaphoreType.DMA((KC,)),
            pltpu.SemaphoreType.DMA((KC,)),
            pltpu.SemaphoreType.DMA((KC,)),
            pltpu.SemaphoreType.DMA((KC,)),
            pltpu.SemaphoreType.DMA((KC,)),
            pltpu.SemaphoreType.DMA((KC,)),
            pltpu.SemaphoreType.DMA((KC,)),
            pltpu.SemaphoreType.DMA((KC,)),
            pltpu.SemaphoreType.DMA((KC,)),
            pltpu.SemaphoreType.DMA((KC,)),
            pltpu.SemaphoreType.DMA,
        ],
        compiler_params=pltpu.CompilerParams(collective_id=0),
    )(x)


# device time: 338453 ns/iter; 2.3034x vs baseline; 1.0301x over previous
import jax
import jax.numpy as jnp
from jax import lax
from jax.experimental import pallas as pl
from jax.experimental.pallas import tpu as pltpu

KC = 16


def kernel(x):
    m, n = x.shape
    blk = m // 4
    cr = blk // KC

    def body(x_ref, out_ref, zrecv_ref, a_ref, o_ref,
             sz_send, sz_recv, sx_send, sx_recv, sy_send, sy_recv,
             srx_send, srx_recv, sry_send, sry_recv, cp_sem):
        my_x = lax.axis_index("x")
        my_y = lax.axis_index("y")
        my_z = lax.axis_index("z")
        z_nbr = (my_x, my_y, 1 - my_z)
        x_nbr = (1 - my_x, my_y, my_z)
        y_nbr = (my_x, 1 - my_y, my_z)

        r = 2 * my_x + my_y
        rx = 2 * (1 - my_x) + my_y
        ry = 2 * my_x + (1 - my_y)
        rd = 2 * (1 - my_x) + (1 - my_y)

        def rows(blk_id, k):
            return pl.ds(blk_id * blk + k * cr, cr)

        barrier = pltpu.get_barrier_semaphore()
        for nbr in (z_nbr, x_nbr, y_nbr):
            pl.semaphore_signal(barrier, inc=1, device_id=nbr,
                                device_id_type=pl.DeviceIdType.MESH)
        pl.semaphore_wait(barrier, 3)

        z_rdmas = []
        for k in range(KC):
            rdma = pltpu.make_async_remote_copy(
                src_ref=x_ref.at[rows(r, k), :],
                dst_ref=zrecv_ref.at[pl.ds(k * cr, cr), :],
                send_sem=sz_send.at[k],
                recv_sem=sz_recv.at[k],
                device_id=z_nbr,
                device_id_type=pl.DeviceIdType.MESH,
            )
            rdma.start()
            z_rdmas.append(rdma)

        def recv_desc(blk_id, k, send_sem, recv_sem, nbr):
            return pltpu.make_async_remote_copy(
                src_ref=out_ref.at[rows(blk_id, k), :],
                dst_ref=out_ref.at[rows(blk_id, k), :],
                send_sem=send_sem.at[k],
                recv_sem=recv_sem.at[k],
                device_id=nbr,
                device_id_type=pl.DeviceIdType.MESH,
            )

        xy_rdmas = []
        for k in range(KC):
            z_rdmas[k].wait_recv()
            cp_a = pltpu.make_async_copy(
                x_ref.at[rows(r, k), :], a_ref, cp_sem)
            cp_a.start()
            cp_a.wait()
            o_ref[...] = a_ref[...] + zrecv_ref[pl.ds(k * cr, cr), :]
            cp_o = pltpu.make_async_copy(
                o_ref, out_ref.at[rows(r, k), :], cp_sem)
            cp_o.start()
            cp_o.wait()
            for send_sem, recv_sem, nbr in (
                (sx_send, sx_recv, x_nbr),
                (sy_send, sy_recv, y_nbr),
            ):
                rdma = pltpu.make_async_remote_copy(
                    src_ref=out_ref.at[rows(r, k), :],
                    dst_ref=out_ref.at[rows(r, k), :],
                    send_sem=send_sem.at[k],
                    recv_sem=recv_sem.at[k],
                    device_id=nbr,
                    device_id_type=pl.DeviceIdType.MESH,
                )
                rdma.start()
                xy_rdmas.append(rdma)

        x_in = [recv_desc(rx, k, sx_send, sx_recv, x_nbr) for k in range(KC)]
        y_in = [recv_desc(ry, k, sy_send, sy_recv, y_nbr) for k in range(KC)]
        d_in = [
            recv_desc(rd, k,
                      srx_send if k % 2 == 0 else sry_send,
                      srx_recv if k % 2 == 0 else sry_recv,
                      x_nbr if k % 2 == 0 else y_nbr)
            for k in range(KC)
        ]
        relays = []
        for k in range(KC):
            if k % 2 == 0:
                y_in[k].wait_recv()
                src_blk, send_sem, recv_sem, nbr = ry, srx_send, srx_recv, x_nbr
            else:
                x_in[k].wait_recv()
                src_blk, send_sem, recv_sem, nbr = rx, sry_send, sry_recv, y_nbr
            rdma = pltpu.make_async_remote_copy(
                src_ref=out_ref.at[rows(src_blk, k), :],
                dst_ref=out_ref.at[rows(src_blk, k), :],
                send_sem=send_sem.at[k],
                recv_sem=recv_sem.at[k],
                device_id=nbr,
                device_id_type=pl.DeviceIdType.MESH,
            )
            rdma.start()
            relays.append(rdma)

        for k in range(KC):
            if k % 2 == 0:
                x_in[k].wait_recv()
            else:
                y_in[k].wait_recv()
            d_in[k].wait_recv()
        for rdma in z_rdmas + xy_rdmas + relays:
            rdma.wait_send()

    return pl.pallas_call(
        body,
        out_shape=jax.ShapeDtypeStruct((m, n), x.dtype),
        in_specs=[pl.BlockSpec(memory_space=pl.ANY)],
        out_specs=pl.BlockSpec(memory_space=pl.ANY),
        scratch_shapes=[
            pltpu.VMEM((blk, n), x.dtype),
            pltpu.VMEM((cr, n), x.dtype),
            pltpu.VMEM((cr, n), x.dtype),
            pltpu.SemaphoreType.DMA((KC,)),
            pltpu.SemaphoreType.DMA((KC,)),
            pltpu.SemaphoreType.DMA((KC,)),
            pltpu.SemaphoreType.DMA((KC,)),
            pltpu.SemaphoreType.DMA((KC,)),
            pltpu.SemaphoreType.DMA((KC,)),
            pltpu.SemaphoreType.DMA((KC,)),
            pltpu.SemaphoreType.DMA((KC,)),
            pltpu.SemaphoreType.DMA((KC,)),
            pltpu.SemaphoreType.DMA((KC,)),
            pltpu.SemaphoreType.DMA,
        ],
        compiler_params=pltpu.CompilerParams(collective_id=0),
    )(x)


# device time: 336001 ns/iter; 2.3202x vs baseline; 1.0073x over previous
import jax
import jax.numpy as jnp
from jax import lax
from jax.experimental import pallas as pl
from jax.experimental.pallas import tpu as pltpu

KC = 16


def kernel(x):
    m, n = x.shape
    blk = m // 4
    cr = blk // KC

    def body(x_ref, out_ref, zrecv_ref, araw_ref, red_ref,
             sz_send, sz_recv, sx_send, sx_recv, sy_send, sy_recv,
             srx_send, srx_recv, sry_send, sry_recv, cp_sem, co_sems):
        my_x = lax.axis_index("x")
        my_y = lax.axis_index("y")
        my_z = lax.axis_index("z")
        z_nbr = (my_x, my_y, 1 - my_z)
        x_nbr = (1 - my_x, my_y, my_z)
        y_nbr = (my_x, 1 - my_y, my_z)

        r = 2 * my_x + my_y
        rx = 2 * (1 - my_x) + my_y
        ry = 2 * my_x + (1 - my_y)
        rd = 2 * (1 - my_x) + (1 - my_y)

        def rows(blk_id, k):
            return pl.ds(blk_id * blk + k * cr, cr)

        barrier = pltpu.get_barrier_semaphore()
        for nbr in (z_nbr, x_nbr, y_nbr):
            pl.semaphore_signal(barrier, inc=1, device_id=nbr,
                                device_id_type=pl.DeviceIdType.MESH)
        pl.semaphore_wait(barrier, 3)

        z_rdmas = []
        for k in range(KC):
            rdma = pltpu.make_async_remote_copy(
                src_ref=x_ref.at[rows(r, k), :],
                dst_ref=zrecv_ref.at[pl.ds(k * cr, cr), :],
                send_sem=sz_send.at[k],
                recv_sem=sz_recv.at[k],
                device_id=z_nbr,
                device_id_type=pl.DeviceIdType.MESH,
            )
            rdma.start()
            z_rdmas.append(rdma)

        cp_raw = pltpu.make_async_copy(
            x_ref.at[pl.ds(r * blk, blk), :], araw_ref, cp_sem)
        cp_raw.start()

        def recv_desc(blk_id, k, send_sem, recv_sem, nbr):
            return pltpu.make_async_remote_copy(
                src_ref=out_ref.at[rows(blk_id, k), :],
                dst_ref=out_ref.at[rows(blk_id, k), :],
                send_sem=send_sem.at[k],
                recv_sem=recv_sem.at[k],
                device_id=nbr,
                device_id_type=pl.DeviceIdType.MESH,
            )

        xy_rdmas = []
        out_cps = []
        for k in range(KC):
            z_rdmas[k].wait_recv()
            if k == 0:
                cp_raw.wait()
            ck = pl.ds(k * cr, cr)
            red_ref[ck, :] = araw_ref[ck, :] + zrecv_ref[ck, :]
            for send_sem, recv_sem, nbr in (
                (sx_send, sx_recv, x_nbr),
                (sy_send, sy_recv, y_nbr),
            ):
                rdma = pltpu.make_async_remote_copy(
                    src_ref=red_ref.at[ck, :],
                    dst_ref=out_ref.at[rows(r, k), :],
                    send_sem=send_sem.at[k],
                    recv_sem=recv_sem.at[k],
                    device_id=nbr,
                    device_id_type=pl.DeviceIdType.MESH,
                )
                rdma.start()
                xy_rdmas.append(rdma)
            cp_o = pltpu.make_async_copy(
                red_ref.at[ck, :], out_ref.at[rows(r, k), :], co_sems.at[k])
            cp_o.start()
            out_cps.append(cp_o)

        x_in = [recv_desc(rx, k, sx_send, sx_recv, x_nbr) for k in range(KC)]
        y_in = [recv_desc(ry, k, sy_send, sy_recv, y_nbr) for k in range(KC)]
        d_in = [
            recv_desc(rd, k,
                      srx_send if k % 2 == 0 else sry_send,
                      srx_recv if k % 2 == 0 else sry_recv,
                      x_nbr if k % 2 == 0 else y_nbr)
            for k in range(KC)
        ]
        relays = []
        for k in range(KC):
            if k % 2 == 0:
                y_in[k].wait_recv()
                src_blk, send_sem, recv_sem, nbr = ry, srx_send, srx_recv, x_nbr
            else:
                x_in[k].wait_recv()
                src_blk, send_sem, recv_sem, nbr = rx, sry_send, sry_recv, y_nbr
            rdma = pltpu.make_async_remote_copy(
                src_ref=out_ref.at[rows(src_blk, k), :],
                dst_ref=out_ref.at[rows(src_blk, k), :],
                send_sem=send_sem.at[k],
                recv_sem=recv_sem.at[k],
                device_id=nbr,
                device_id_type=pl.DeviceIdType.MESH,
            )
            rdma.start()
            relays.append(rdma)

        for k in range(KC):
            if k % 2 == 0:
                x_in[k].wait_recv()
            else:
                y_in[k].wait_recv()
            d_in[k].wait_recv()
        for rdma in z_rdmas + xy_rdmas + relays:
            rdma.wait_send()
        for cp in out_cps:
            cp.wait()

    return pl.pallas_call(
        body,
        out_shape=jax.ShapeDtypeStruct((m, n), x.dtype),
        in_specs=[pl.BlockSpec(memory_space=pl.ANY)],
        out_specs=pl.BlockSpec(memory_space=pl.ANY),
        scratch_shapes=[
            pltpu.VMEM((blk, n), x.dtype),
            pltpu.VMEM((blk, n), x.dtype),
            pltpu.VMEM((blk, n), x.dtype),
            pltpu.SemaphoreType.DMA((KC,)),
            pltpu.SemaphoreType.DMA((KC,)),
            pltpu.SemaphoreType.DMA((KC,)),
            pltpu.SemaphoreType.DMA((KC,)),
            pltpu.SemaphoreType.DMA((KC,)),
            pltpu.SemaphoreType.DMA((KC,)),
            pltpu.SemaphoreType.DMA((KC,)),
            pltpu.SemaphoreType.DMA((KC,)),
            pltpu.SemaphoreType.DMA((KC,)),
            pltpu.SemaphoreType.DMA((KC,)),
            pltpu.SemaphoreType.DMA,
            pltpu.SemaphoreType.DMA((KC,)),
        ],
        compiler_params=pltpu.CompilerParams(
            collective_id=0, vmem_limit_bytes=100 * 1024 * 1024),
    )(x)


# device time: 335177 ns/iter; 2.3259x vs baseline; 1.0025x over previous
import jax
import jax.numpy as jnp
from jax import lax
from jax.experimental import pallas as pl
from jax.experimental.pallas import tpu as pltpu

KC = 16
CW = 896
FW = 128


def kernel(x):
    m, n = x.shape
    blk = m // 4
    cr = blk // KC

    def body(x_ref, out_ref, zrecv_ref, araw_ref, red_ref,
             sz_send, sz_recv, sx_send, sx_recv, sy_send, sy_recv,
             srx_send, srx_recv, sry_send, sry_recv,
             sfx_send, sfx_recv, sfy_send, sfy_recv, cp_sem, co_sems):
        my_x = lax.axis_index("x")
        my_y = lax.axis_index("y")
        my_z = lax.axis_index("z")
        z_nbr = (my_x, my_y, 1 - my_z)
        x_nbr = (1 - my_x, my_y, my_z)
        y_nbr = (my_x, 1 - my_y, my_z)

        r = 2 * my_x + my_y
        rx = 2 * (1 - my_x) + my_y
        ry = 2 * my_x + (1 - my_y)
        rd = 2 * (1 - my_x) + (1 - my_y)

        dir_c = my_z * FW
        fwd_in_c = (1 - my_z) * CW
        fwd_out_c = my_z * CW

        def rows(blk_id, k):
            return pl.ds(blk_id * blk + k * cr, cr)

        barrier = pltpu.get_barrier_semaphore()
        for nbr in (z_nbr, x_nbr, y_nbr):
            pl.semaphore_signal(barrier, inc=1, device_id=nbr,
                                device_id_type=pl.DeviceIdType.MESH)
        pl.semaphore_wait(barrier, 3)

        z_rdmas = []
        for k in range(KC):
            rdma = pltpu.make_async_remote_copy(
                src_ref=x_ref.at[rows(r, k), :],
                dst_ref=zrecv_ref.at[pl.ds(k * cr, cr), :],
                send_sem=sz_send.at[k],
                recv_sem=sz_recv.at[k],
                device_id=z_nbr,
                device_id_type=pl.DeviceIdType.MESH,
            )
            rdma.start()
            z_rdmas.append(rdma)

        cp_raw = pltpu.make_async_copy(
            x_ref.at[pl.ds(r * blk, blk), :], araw_ref, cp_sem)
        cp_raw.start()

        def recv_desc(blk_id, k, col, w, send_sem, recv_sem, nbr):
            return pltpu.make_async_remote_copy(
                src_ref=out_ref.at[rows(blk_id, k), pl.ds(col, w)],
                dst_ref=out_ref.at[rows(blk_id, k), pl.ds(col, w)],
                send_sem=send_sem.at[k],
                recv_sem=recv_sem.at[k],
                device_id=nbr,
                device_id_type=pl.DeviceIdType.MESH,
            )

        xy_rdmas = []
        out_cps = []
        for k in range(KC):
            z_rdmas[k].wait_recv()
            if k == 0:
                cp_raw.wait()
            ck = pl.ds(k * cr, cr)
            red_ref[ck, :] = araw_ref[ck, :] + zrecv_ref[ck, :]
            for send_sem, recv_sem, nbr in (
                (sx_send, sx_recv, x_nbr),
                (sy_send, sy_recv, y_nbr),
            ):
                rdma = pltpu.make_async_remote_copy(
                    src_ref=red_ref.at[ck, pl.ds(dir_c, CW)],
                    dst_ref=out_ref.at[rows(r, k), pl.ds(dir_c, CW)],
                    send_sem=send_sem.at[k],
                    recv_sem=recv_sem.at[k],
                    device_id=nbr,
                    device_id_type=pl.DeviceIdType.MESH,
                )
                rdma.start()
                xy_rdmas.append(rdma)
            cp_o = pltpu.make_async_copy(
                red_ref.at[ck, :], out_ref.at[rows(r, k), :], co_sems.at[k])
            cp_o.start()
            out_cps.append(cp_o)

        x_in = [recv_desc(rx, k, dir_c, CW, sx_send, sx_recv, x_nbr)
                for k in range(KC)]
        y_in = [recv_desc(ry, k, dir_c, CW, sy_send, sy_recv, y_nbr)
                for k in range(KC)]
        zfx_in = [recv_desc(rx, k, fwd_in_c, FW, sfx_send, sfx_recv, z_nbr)
                  for k in range(KC)]
        zfy_in = [recv_desc(ry, k, fwd_in_c, FW, sfy_send, sfy_recv, z_nbr)
                  for k in range(KC)]
        d_in = [
            recv_desc(rd, k, 0, n,
                      srx_send if k % 2 == 0 else sry_send,
                      srx_recv if k % 2 == 0 else sry_recv,
                      x_nbr if k % 2 == 0 else y_nbr)
            for k in range(KC)
        ]
        relays = []
        fwds = []
        for k in range(KC):
            x_in[k].wait_recv()
            fwd = pltpu.make_async_remote_copy(
                src_ref=out_ref.at[rows(rx, k), pl.ds(fwd_out_c, FW)],
                dst_ref=out_ref.at[rows(rx, k), pl.ds(fwd_out_c, FW)],
                send_sem=sfx_send.at[k],
                recv_sem=sfx_recv.at[k],
                device_id=z_nbr,
                device_id_type=pl.DeviceIdType.MESH,
            )
            fwd.start()
            fwds.append(fwd)
            if k % 2 == 1:
                zfx_in[k].wait_recv()
                rdma = pltpu.make_async_remote_copy(
                    src_ref=out_ref.at[rows(rx, k), :],
                    dst_ref=out_ref.at[rows(rx, k), :],
                    send_sem=sry_send.at[k],
                    recv_sem=sry_recv.at[k],
                    device_id=y_nbr,
                    device_id_type=pl.DeviceIdType.MESH,
                )
                rdma.start()
                relays.append(rdma)
            y_in[k].wait_recv()
            fwd = pltpu.make_async_remote_copy(
                src_ref=out_ref.at[rows(ry, k), pl.ds(fwd_out_c, FW)],
                dst_ref=out_ref.at[rows(ry, k), pl.ds(fwd_out_c, FW)],
                send_sem=sfy_send.at[k],
                recv_sem=sfy_recv.at[k],
                device_id=z_nbr,
                device_id_type=pl.DeviceIdType.MESH,
            )
            fwd.start()
            fwds.append(fwd)
            if k % 2 == 0:
                zfy_in[k].wait_recv()
                rdma = pltpu.make_async_remote_copy(
                    src_ref=out_ref.at[rows(ry, k), :],
                    dst_ref=out_ref.at[rows(ry, k), :],
                    send_sem=srx_send.at[k],
                    recv_sem=srx_recv.at[k],
                    device_id=x_nbr,
                    device_id_type=pl.DeviceIdType.MESH,
                )
                rdma.start()
                relays.append(rdma)

        for k in range(KC):
            if k % 2 == 0:
                zfx_in[k].wait_recv()
            else:
                zfy_in[k].wait_recv()
            d_in[k].wait_recv()
        for rdma in z_rdmas + xy_rdmas + relays + fwds:
            rdma.wait_send()
        for cp in out_cps:
            cp.wait()

    return pl.pallas_call(
        body,
        out_shape=jax.ShapeDtypeStruct((m, n), x.dtype),
        in_specs=[pl.BlockSpec(memory_space=pl.ANY)],
        out_specs=pl.BlockSpec(memory_space=pl.ANY),
        scratch_shapes=[
            pltpu.VMEM((blk, n), x.dtype),
            pltpu.VMEM((blk, n), x.dtype),
            pltpu.VMEM((blk, n), x.dtype),
            pltpu.SemaphoreType.DMA((KC,)),
            pltpu.SemaphoreType.DMA((KC,)),
            pltpu.SemaphoreType.DMA((KC,)),
            pltpu.SemaphoreType.DMA((KC,)),
            pltpu.SemaphoreType.DMA((KC,)),
            pltpu.SemaphoreType.DMA((KC,)),
            pltpu.SemaphoreType.DMA((KC,)),
            pltpu.SemaphoreType.DMA((KC,)),
            pltpu.SemaphoreType.DMA((KC,)),
            pltpu.SemaphoreType.DMA((KC,)),
            pltpu.SemaphoreType.DMA((KC,)),
            pltpu.SemaphoreType.DMA((KC,)),
            pltpu.SemaphoreType.DMA((KC,)),
            pltpu.SemaphoreType.DMA((KC,)),
            pltpu.SemaphoreType.DMA,
            pltpu.SemaphoreType.DMA((KC,)),
        ],
        compiler_params=pltpu.CompilerParams(
            collective_id=0, vmem_limit_bytes=100 * 1024 * 1024),
    )(x)
